# baseline (speedup 1.0000x reference)
import functools

import jax
import jax.numpy as jnp
import numpy as np
from jax import lax
from jax.experimental import pallas as pl
from jax.experimental.pallas import tpu as pltpu
from jax.experimental.pallas import tpu_sc as plsc

K = 16
GROUPS = 8
BLK = 128
NB = 128
NC, NS = 2, 16
NW = NC * NS
QPW = 128
CQ = 4
ROWS = CQ * K
NCHUNK = QPW // CQ
INF = np.float32(np.inf)


def _d2_body(q_ref, x_ref, d2_ref, bm_ref):
    q = q_ref[0]
    x = x_ref[0]
    qn = jnp.sum(q * q, axis=1)
    pn = jnp.sum(x * x, axis=1)
    cross = lax.dot_general(q, x, (((1,), (1,)), ((), ())),
                            preferred_element_type=jnp.float32)
    d2 = qn[:, None] + pn[None, :] - 2.0 * cross
    d2_ref[0] = d2
    tm, tn = d2.shape
    bm_ref[0, 0] = jnp.min(d2.reshape(tm, tn // BLK, BLK), axis=-1)


def _d2_and_blockmins(coords, xyz):
    B, M, _ = coords.shape
    N = xyz.shape[1]
    TM, TN = 256, 2048
    return pl.pallas_call(
        _d2_body,
        grid=(B, M // TM, N // TN),
        in_specs=[
            pl.BlockSpec((1, TM, 3), lambda b, i, j: (b, i, 0)),
            pl.BlockSpec((1, TN, 3), lambda b, i, j: (b, j, 0)),
        ],
        out_specs=[
            pl.BlockSpec((1, TM, TN), lambda b, i, j: (b, i, j)),
            pl.BlockSpec((1, 1, TM, TN // BLK), lambda b, i, j: (b, j, i, 0)),
        ],
        out_shape=[
            jax.ShapeDtypeStruct((B, M, N), jnp.float32),
            jax.ShapeDtypeStruct((B, N // TN, M, TN // BLK), jnp.float32),
        ],
    )(coords, xyz)


def _bcast(ref1d, idx):
    return plsc.load_gather(ref1d, [jnp.full((16,), idx, jnp.int32)])


def _merge16(ak, av, bk, bv):
    rbk = lax.rev(bk, (0,))
    rbv = lax.rev(bv, (0,))
    m = (ak < rbk) | ((ak == rbk) & (av <= rbv))
    mk = jnp.where(m, ak, rbk)
    mv = jnp.where(m, av, rbv)
    sk, sv = plsc.sort_key_val(mk, mv)
    return sk, sv


def _tie_fix(k, v, partner, iota):
    kp = k.at[partner].get(mode='promise_in_bounds')
    vp = v.at[partner].get(mode='promise_in_bounds')
    tie = k == kp
    nv = jnp.where(tie & (iota < partner), jnp.minimum(v, vp),
                   jnp.where(tie & (iota > partner), jnp.maximum(v, vp), v))
    return nv


def _sc_select_gather(bm4, d2r, xyzT, coordflat, fvecf, BM, N):
    mesh = plsc.VectorSubcoreMesh(core_axis_name="c", subcore_axis_name="s")

    @functools.partial(
        pl.kernel,
        out_type=[
            jax.ShapeDtypeStruct((BM * K, 256), jnp.float32),
            jax.ShapeDtypeStruct((BM * K * 16,), jnp.float32),
        ],
        mesh=mesh,
        compiler_params=pltpu.CompilerParams(needs_layout_passes=False),
        scratch_types=[
            pltpu.VMEM((8 * (QPW // 2) * 16,), jnp.float32),
            pltpu.VMEM((QPW * 3,), jnp.float32),
            pltpu.VMEM((QPW * K,), jnp.int32),
            pltpu.SMEM((QPW,), jnp.float32),
            pltpu.VMEM((2, ROWS, BLK), jnp.float32),
            pltpu.VMEM((ROWS,), jnp.int32),
            pltpu.VMEM((ROWS, 256), jnp.float32),
            pltpu.VMEM((N,), jnp.float32),
            pltpu.VMEM((N,), jnp.float32),
            pltpu.VMEM((N,), jnp.float32),
            pltpu.VMEM((ROWS * 16,), jnp.float32),
            pltpu.SemaphoreType.DMA((2,)),
            pltpu.SemaphoreType.DMA,
        ],
    )
    def body(bm_hbm, d2r_hbm, xyzT_hbm, cp_hbm, fv_hbm, gath_hbm, xout_hbm,
             bmv, cpflat, rowflat, thrv, candv, fidxv, fbufv, px, py, pz, xov,
             cand_sem, fg_sem):
        wid = lax.axis_index("s") * NC + lax.axis_index("c")
        qbase = wid * QPW
        b = qbase // 2048
        mq = qbase - b * 2048
        nbase = b * N
        iota = lax.iota(jnp.int32, 16)

        pltpu.sync_copy(cp_hbm.at[pl.ds(qbase * 3, QPW * 3)], cpflat)
        pltpu.sync_copy(xyzT_hbm.at[pl.ds((b * 3 + 0) * N, N)], px)
        pltpu.sync_copy(xyzT_hbm.at[pl.ds((b * 3 + 1) * N, N)], py)
        pltpu.sync_copy(xyzT_hbm.at[pl.ds((b * 3 + 2) * N, N)], pz)

        for h in range(2):
            hq = h * (QPW // 2)
            for r in range(8):
                pltpu.sync_copy(
                    bm_hbm.at[pl.ds(((b * 8 + r) * 2048 + mq + hq) * 16,
                                    (QPW // 2) * 16)],
                    bmv.at[pl.ds(r * (QPW // 2) * 16, (QPW // 2) * 16)])

            def p1(q, _, hq=hq):
                lq = hq + q
                ks, vs = [], []
                for r in range(8):
                    k = bmv[pl.ds((r * (QPW // 2) + q) * 16, 16)]
                    k, v = plsc.sort_key_val(k, iota + r * 16)
                    ks.append(k)
                    vs.append(v)
                while len(ks) > 1:
                    nk, nv = [], []
                    for i in range(0, len(ks), 2):
                        mk, mv = _merge16(ks[i], vs[i], ks[i + 1], vs[i + 1])
                        nk.append(mk)
                        nv.append(mv)
                    ks, vs = nk, nv
                thrv[lq] = jnp.max(ks[0])
                sid, _ = plsc.sort_key_val(vs[0], vs[0])
                rowflat[pl.ds(lq * 16, 16)] = sid + (qbase + lq) * NB
                return 0

            lax.fori_loop(0, QPW // 2, p1, 0)

        def cand_fetch(chunk, slot):
            return pltpu.async_copy(
                d2r_hbm.at[rowflat.at[pl.ds(chunk * ROWS, ROWS)]],
                candv.at[slot], cand_sem.at[slot])

        cand_fetch(0, 0)
        cand_fetch(1, 1)

        def chunk_body(p, _):
            for s in range(2):
                c = p * 2 + s
                pltpu.make_async_copy(
                    d2r_hbm.at[pl.ds(0, ROWS)], candv.at[s],
                    cand_sem.at[s]).wait()

                def q_body(q2, _):
                    lq = c * CQ + q2
                    theta = thrv[lq]
                    qg128 = (qbase + lq) * NB

                    def bi_body(bi, cur):
                        ck, cv = cur
                        blkb = plsc.load_gather(
                            rowflat,
                            [jnp.full((16,), lq * 16 + bi, jnp.int32)])
                        row = q2 * 16 + bi
                        for part in range(8):
                            v = candv[s, row, pl.ds(part * 16, 16)]
                            msk = v <= theta
                            vidx = (blkb - qg128) * BLK + part * 16 + iota

                            def do(ck, cv, v, vidx, msk):
                                sk, sv = plsc.sort_key_val(
                                    jnp.where(msk, v, INF), vidx)
                                return _merge16(ck, cv, sk, sv)

                            ck, cv = lax.cond(
                                jnp.any(msk), do,
                                lambda ck, cv, v, vidx, msk: (ck, cv),
                                ck, cv, v, vidx, msk)
                        return ck, cv

                    fin_k, fin_v = lax.fori_loop(
                        0, 16, bi_body,
                        (jnp.full((16,), INF, jnp.float32),
                         jnp.zeros((16,), jnp.int32)))
                    p_even = lax.bitwise_xor(iota, 1)
                    p_odd = jnp.clip(lax.bitwise_xor(iota - 1, 1) + 1, 0, 15)
                    for _ in range(2):
                        fin_v = _tie_fix(fin_k, fin_v, p_even, iota)
                        fin_v = _tie_fix(fin_k, fin_v, p_odd, iota)
                    fidxv[pl.ds(q2 * 16, 16)] = fin_v + nbase

                    nx = plsc.load_gather(px, [fin_v])
                    ny = plsc.load_gather(py, [fin_v])
                    nz = plsc.load_gather(pz, [fin_v])
                    rowv = (q2 * 16 + iota) * 16
                    cols = [nx - _bcast(cpflat, lq * 3 + 0),
                            ny - _bcast(cpflat, lq * 3 + 1),
                            nz - _bcast(cpflat, lq * 3 + 2),
                            nx, ny, nz]
                    for j, colv in enumerate(cols):
                        plsc.store_scatter(xov, [rowv + j], colv)
                    return 0

                lax.fori_loop(0, CQ, q_body, 0)

                pltpu.async_copy(fv_hbm.at[fidxv], fbufv, fg_sem)

                obase = (qbase + c * CQ) * K
                pltpu.make_async_copy(
                    fv_hbm.at[pl.ds(0, ROWS)], fbufv, fg_sem).wait()
                pltpu.sync_copy(fbufv, gath_hbm.at[pl.ds(obase, ROWS)])
                pltpu.sync_copy(xov, xout_hbm.at[pl.ds(obase * 16, ROWS * 16)])

                cand_fetch(jnp.minimum(c + 2, NCHUNK - 1), s)
            return 0

        lax.fori_loop(0, NCHUNK // 2, chunk_body, 0)
        for s in range(2):
            pltpu.make_async_copy(
                d2r_hbm.at[pl.ds(0, ROWS)], candv.at[s], cand_sem.at[s]).wait()

    return body(bm4, d2r, xyzT, coordflat, fvecf)


def _corr_body(g_ref, t_ref, p_ref, x_ref, o_ref):
    g = g_ref[...]
    t = t_ref[...]
    tq = t.shape[0]
    trep = jnp.broadcast_to(t[:, None, :], (tq, K, 256)).reshape(tq * K, 256)
    z = g * trep
    corr = jnp.dot(z, p_ref[...], preferred_element_type=jnp.float32) \
        / np.float32((256.0 / GROUPS) ** 0.5)
    o_ref[...] = jnp.concatenate([corr, x_ref[...][:, :6]], axis=1)


def _corr(gath, targets_flat, pmask, xout):
    R = gath.shape[0]
    TQ = 128
    return pl.pallas_call(
        _corr_body,
        grid=(R // (TQ * K),),
        in_specs=[
            pl.BlockSpec((TQ * K, 256), lambda i: (i, 0)),
            pl.BlockSpec((TQ, 256), lambda i: (i, 0)),
            pl.BlockSpec((256, GROUPS), lambda i: (0, 0)),
            pl.BlockSpec((TQ * K, 16), lambda i: (i, 0)),
        ],
        out_specs=pl.BlockSpec((TQ * K, GROUPS + 6), lambda i: (i, 0)),
        out_shape=jax.ShapeDtypeStruct((R, GROUPS + 6), jnp.float32),
    )(gath, targets_flat, pmask, xout)


def kernel(xyz, fvec, targets, coords_world_xyz):
    B, N, C = fvec.shape
    M = targets.shape[1]
    BM = B * M

    xyzT = jnp.transpose(xyz, (0, 2, 1))
    d2, bmins4 = _d2_and_blockmins(coords_world_xyz, xyz)

    d2r = d2.reshape(BM * NB, BLK)
    coordflat = coords_world_xyz.reshape(BM * 3)
    fvecf = fvec.reshape(B * N, C)

    xyzTflat = xyzT.reshape(B * 3 * N)
    bmflat = bmins4.reshape(B * 8 * M * 16)
    gath, xoutf = _sc_select_gather(bmflat, d2r, xyzTflat, coordflat, fvecf,
                                    BM, N)
    xout = xoutf.reshape(BM * K, 16)

    pmask = jnp.asarray(
        (np.arange(256)[:, None] // (C // GROUPS)
         == np.arange(GROUPS)[None, :]).astype(np.float32))
    out = _corr(gath, targets.reshape(BM, C), pmask, xout)
    return out.reshape(B, M, K, GROUPS + 6)


# speedup vs baseline: 1.4498x; 1.4498x over previous
import functools

import jax
import jax.numpy as jnp
import numpy as np
from jax import lax
from jax.experimental import pallas as pl
from jax.experimental.pallas import tpu as pltpu
from jax.experimental.pallas import tpu_sc as plsc

K = 16
GROUPS = 8
BLK = 128
NB = 128
NC, NS = 2, 16
NW = NC * NS
QPW = 128
CQ = 4
ROWS = CQ * K
NCHUNK = QPW // CQ
INF = np.float32(np.inf)


def _d2_body(q_ref, x_ref, d2_ref, bm_ref):
    q = q_ref[0]
    x = x_ref[0]
    qn = jnp.sum(q * q, axis=1)
    pn = jnp.sum(x * x, axis=1)
    cross = lax.dot_general(q, x, (((1,), (1,)), ((), ())),
                            preferred_element_type=jnp.float32)
    d2 = qn[:, None] + pn[None, :] - 2.0 * cross
    tm, tn = d2.shape
    d2b = d2.reshape(tm, tn // BLK, BLK)
    d2_ref[0] = d2b
    bm_ref[0, 0] = jnp.min(d2b, axis=-1).reshape(tm // 8, 8 * (tn // BLK))


def _d2_and_blockmins(coords, xyz):
    B, M, _ = coords.shape
    N = xyz.shape[1]
    TM, TN = 256, 2048
    return pl.pallas_call(
        _d2_body,
        grid=(B, M // TM, N // TN),
        in_specs=[
            pl.BlockSpec((1, TM, 3), lambda b, i, j: (b, i, 0)),
            pl.BlockSpec((1, TN, 3), lambda b, i, j: (b, j, 0)),
        ],
        out_specs=[
            pl.BlockSpec((1, TM, TN // BLK, BLK), lambda b, i, j: (b, i, j, 0)),
            pl.BlockSpec((1, 1, TM // 8, 8 * (TN // BLK)),
                         lambda b, i, j: (b, j, i, 0)),
        ],
        out_shape=[
            jax.ShapeDtypeStruct((B, M, N // BLK, BLK), jnp.float32),
            jax.ShapeDtypeStruct((B, N // TN, M // 8, 8 * (TN // BLK)),
                                 jnp.float32),
        ],
    )(coords, xyz)


def _bcast(ref1d, idx):
    return plsc.load_gather(ref1d, [jnp.full((16,), idx, jnp.int32)])


def _merge16(ak, av, bk, bv):
    rbk = lax.rev(bk, (0,))
    rbv = lax.rev(bv, (0,))
    m = (ak < rbk) | ((ak == rbk) & (av <= rbv))
    mk = jnp.where(m, ak, rbk)
    mv = jnp.where(m, av, rbv)
    sk, sv = plsc.sort_key_val(mk, mv)
    return sk, sv


def _tie_fix(k, v, partner, iota):
    kp = k.at[partner].get(mode='promise_in_bounds')
    vp = v.at[partner].get(mode='promise_in_bounds')
    tie = k == kp
    nv = jnp.where(tie & (iota < partner), jnp.minimum(v, vp),
                   jnp.where(tie & (iota > partner), jnp.maximum(v, vp), v))
    return nv


def _sc_select_gather(bm4, d2r, xyzT, coordflat, fvecf, BM, N):
    mesh = plsc.VectorSubcoreMesh(core_axis_name="c", subcore_axis_name="s")

    @functools.partial(
        pl.kernel,
        out_type=[
            jax.ShapeDtypeStruct((BM * K, 256), jnp.float32),
            jax.ShapeDtypeStruct((BM * K * 16,), jnp.float32),
        ],
        mesh=mesh,
        compiler_params=pltpu.CompilerParams(needs_layout_passes=False),
        scratch_types=[
            pltpu.VMEM((8 * (QPW // 2) * 16,), jnp.float32),
            pltpu.VMEM((QPW * 3,), jnp.float32),
            pltpu.VMEM((QPW * K,), jnp.int32),
            pltpu.SMEM((QPW,), jnp.float32),
            pltpu.VMEM((2, ROWS, BLK), jnp.float32),
            pltpu.VMEM((ROWS,), jnp.int32),
            pltpu.VMEM((ROWS, 256), jnp.float32),
            pltpu.VMEM((N,), jnp.float32),
            pltpu.VMEM((N,), jnp.float32),
            pltpu.VMEM((N,), jnp.float32),
            pltpu.VMEM((ROWS * 16,), jnp.float32),
            pltpu.SemaphoreType.DMA((2,)),
            pltpu.SemaphoreType.DMA,
        ],
    )
    def body(bm_hbm, d2r_hbm, xyzT_hbm, cp_hbm, fv_hbm, gath_hbm, xout_hbm,
             bmv, cpflat, rowflat, thrv, candv, fidxv, fbufv, px, py, pz, xov,
             cand_sem, fg_sem):
        wid = lax.axis_index("s") * NC + lax.axis_index("c")
        qbase = wid * QPW
        b = qbase // 2048
        mq = qbase - b * 2048
        nbase = b * N
        iota = lax.iota(jnp.int32, 16)

        pltpu.sync_copy(cp_hbm.at[pl.ds(qbase * 3, QPW * 3)], cpflat)
        pltpu.sync_copy(xyzT_hbm.at[pl.ds((b * 3 + 0) * N, N)], px)
        pltpu.sync_copy(xyzT_hbm.at[pl.ds((b * 3 + 1) * N, N)], py)
        pltpu.sync_copy(xyzT_hbm.at[pl.ds((b * 3 + 2) * N, N)], pz)

        for h in range(2):
            hq = h * (QPW // 2)
            for r in range(8):
                pltpu.sync_copy(
                    bm_hbm.at[pl.ds(((b * 8 + r) * 2048 + mq + hq) * 16,
                                    (QPW // 2) * 16)],
                    bmv.at[pl.ds(r * (QPW // 2) * 16, (QPW // 2) * 16)])

            def p1(q, _, hq=hq):
                lq = hq + q
                ks, vs = [], []
                for r in range(8):
                    k = bmv[pl.ds((r * (QPW // 2) + q) * 16, 16)]
                    k, v = plsc.sort_key_val(k, iota + r * 16)
                    ks.append(k)
                    vs.append(v)
                while len(ks) > 1:
                    nk, nv = [], []
                    for i in range(0, len(ks), 2):
                        mk, mv = _merge16(ks[i], vs[i], ks[i + 1], vs[i + 1])
                        nk.append(mk)
                        nv.append(mv)
                    ks, vs = nk, nv
                thrv[lq] = jnp.max(ks[0])
                sid, _ = plsc.sort_key_val(vs[0], vs[0])
                rowflat[pl.ds(lq * 16, 16)] = sid + (qbase + lq) * NB
                return 0

            lax.fori_loop(0, QPW // 2, p1, 0)

        def cand_fetch(chunk, slot):
            return pltpu.async_copy(
                d2r_hbm.at[rowflat.at[pl.ds(chunk * ROWS, ROWS)]],
                candv.at[slot], cand_sem.at[slot])

        cand_fetch(0, 0)
        cand_fetch(1, 1)

        def chunk_body(p, _):
            for s in range(2):
                c = p * 2 + s
                pltpu.make_async_copy(
                    d2r_hbm.at[pl.ds(0, ROWS)], candv.at[s],
                    cand_sem.at[s]).wait()

                def q_body(q2, _):
                    lq = c * CQ + q2
                    theta = thrv[lq]
                    qg128 = (qbase + lq) * NB

                    def bi_body(bi, cur):
                        ck, cv = cur
                        blkb = plsc.load_gather(
                            rowflat,
                            [jnp.full((16,), lq * 16 + bi, jnp.int32)])
                        row = q2 * 16 + bi
                        for part in range(8):
                            v = candv[s, row, pl.ds(part * 16, 16)]
                            msk = v <= theta
                            vidx = (blkb - qg128) * BLK + part * 16 + iota

                            def do(ck, cv, v, vidx, msk):
                                sk, sv = plsc.sort_key_val(
                                    jnp.where(msk, v, INF), vidx)
                                return _merge16(ck, cv, sk, sv)

                            ck, cv = lax.cond(
                                jnp.any(msk), do,
                                lambda ck, cv, v, vidx, msk: (ck, cv),
                                ck, cv, v, vidx, msk)
                        return ck, cv

                    fin_k, fin_v = lax.fori_loop(
                        0, 16, bi_body,
                        (jnp.full((16,), INF, jnp.float32),
                         jnp.zeros((16,), jnp.int32)))
                    p_even = lax.bitwise_xor(iota, 1)
                    p_odd = jnp.clip(lax.bitwise_xor(iota - 1, 1) + 1, 0, 15)
                    for _ in range(2):
                        fin_v = _tie_fix(fin_k, fin_v, p_even, iota)
                        fin_v = _tie_fix(fin_k, fin_v, p_odd, iota)
                    fidxv[pl.ds(q2 * 16, 16)] = fin_v + nbase

                    nx = plsc.load_gather(px, [fin_v])
                    ny = plsc.load_gather(py, [fin_v])
                    nz = plsc.load_gather(pz, [fin_v])
                    rowv = (q2 * 16 + iota) * 16
                    cols = [nx - _bcast(cpflat, lq * 3 + 0),
                            ny - _bcast(cpflat, lq * 3 + 1),
                            nz - _bcast(cpflat, lq * 3 + 2),
                            nx, ny, nz]
                    for j, colv in enumerate(cols):
                        plsc.store_scatter(xov, [rowv + j], colv)
                    return 0

                lax.fori_loop(0, CQ, q_body, 0)

                pltpu.async_copy(fv_hbm.at[fidxv], fbufv, fg_sem)

                obase = (qbase + c * CQ) * K
                pltpu.make_async_copy(
                    fv_hbm.at[pl.ds(0, ROWS)], fbufv, fg_sem).wait()
                pltpu.sync_copy(fbufv, gath_hbm.at[pl.ds(obase, ROWS)])
                pltpu.sync_copy(xov, xout_hbm.at[pl.ds(obase * 16, ROWS * 16)])

                cand_fetch(jnp.minimum(c + 2, NCHUNK - 1), s)
            return 0

        lax.fori_loop(0, NCHUNK // 2, chunk_body, 0)
        for s in range(2):
            pltpu.make_async_copy(
                d2r_hbm.at[pl.ds(0, ROWS)], candv.at[s], cand_sem.at[s]).wait()

    return body(bm4, d2r, xyzT, coordflat, fvecf)


def _corr_body(g_ref, t_ref, p_ref, x_ref, o_ref):
    g = g_ref[...]
    t = t_ref[...]
    tq = t.shape[0]
    trep = jnp.broadcast_to(t[:, None, :], (tq, K, 256)).reshape(tq * K, 256)
    z = g * trep
    corr = jnp.dot(z, p_ref[...], preferred_element_type=jnp.float32) \
        / np.float32((256.0 / GROUPS) ** 0.5)
    o_ref[...] = jnp.concatenate([corr, x_ref[...][:, :6]], axis=1)


def _corr(gath, targets_flat, pmask, xout):
    R = gath.shape[0]
    TQ = 128
    return pl.pallas_call(
        _corr_body,
        grid=(R // (TQ * K),),
        in_specs=[
            pl.BlockSpec((TQ * K, 256), lambda i: (i, 0)),
            pl.BlockSpec((TQ, 256), lambda i: (i, 0)),
            pl.BlockSpec((256, GROUPS), lambda i: (0, 0)),
            pl.BlockSpec((TQ * K, 16), lambda i: (i, 0)),
        ],
        out_specs=pl.BlockSpec((TQ * K, GROUPS + 6), lambda i: (i, 0)),
        out_shape=jax.ShapeDtypeStruct((R, GROUPS + 6), jnp.float32),
    )(gath, targets_flat, pmask, xout)


def kernel(xyz, fvec, targets, coords_world_xyz):
    B, N, C = fvec.shape
    M = targets.shape[1]
    BM = B * M

    xyzT = jnp.transpose(xyz, (0, 2, 1))
    d2, bmins4 = _d2_and_blockmins(coords_world_xyz, xyz)

    d2r = d2.reshape(BM * NB, BLK)
    coordflat = coords_world_xyz.reshape(BM * 3)
    fvecf = fvec.reshape(B * N, C)

    xyzTflat = xyzT.reshape(B * 3 * N)
    bmflat = bmins4.reshape(B * 8 * M * 16)
    gath, xoutf = _sc_select_gather(bmflat, d2r, xyzTflat, coordflat, fvecf,
                                    BM, N)

    pmask = jnp.asarray(
        (np.arange(256)[:, None] // (C // GROUPS)
         == np.arange(GROUPS)[None, :]).astype(np.float32))
    out = _corr(gath, targets.reshape(BM, C), pmask,
                xoutf.reshape(BM * K, 16))
    return out.reshape(B, M, K, GROUPS + 6)


# speedup vs baseline: 1.4878x; 1.0262x over previous
import functools

import jax
import jax.numpy as jnp
import numpy as np
from jax import lax
from jax.experimental import pallas as pl
from jax.experimental.pallas import tpu as pltpu
from jax.experimental.pallas import tpu_sc as plsc

K = 16
GROUPS = 8
BLK = 128
NB = 128
NC, NS = 2, 16
NW = NC * NS
QPW = 128
CQ = 4
ROWS = CQ * K
NCHUNK = QPW // CQ
INF = np.float32(np.inf)


def _d2_body(q_ref, x_ref, d2_ref, bm_ref):
    q = q_ref[0]
    x = x_ref[0]
    qn = jnp.sum(q * q, axis=1)
    pn = jnp.sum(x * x, axis=1)
    cross = lax.dot_general(q, x, (((1,), (1,)), ((), ())),
                            preferred_element_type=jnp.float32)
    d2 = qn[:, None] + pn[None, :] - 2.0 * cross
    tm, tn = d2.shape
    d2b = d2.reshape(tm, tn // BLK, BLK)
    d2_ref[0] = d2b
    bm_ref[0, 0] = jnp.min(d2b, axis=-1).reshape(tm // 8, 8 * (tn // BLK))


def _d2_and_blockmins(coords, xyz):
    B, M, _ = coords.shape
    N = xyz.shape[1]
    TM, TN = 256, 2048
    return pl.pallas_call(
        _d2_body,
        grid=(B, M // TM, N // TN),
        in_specs=[
            pl.BlockSpec((1, TM, 3), lambda b, i, j: (b, i, 0)),
            pl.BlockSpec((1, TN, 3), lambda b, i, j: (b, j, 0)),
        ],
        out_specs=[
            pl.BlockSpec((1, TM, TN // BLK, BLK), lambda b, i, j: (b, i, j, 0)),
            pl.BlockSpec((1, 1, TM // 8, 8 * (TN // BLK)),
                         lambda b, i, j: (b, j, i, 0)),
        ],
        out_shape=[
            jax.ShapeDtypeStruct((B, M, N // BLK, BLK), jnp.float32),
            jax.ShapeDtypeStruct((B, N // TN, M // 8, 8 * (TN // BLK)),
                                 jnp.float32),
        ],
    )(coords, xyz)


def _bcast(ref1d, idx):
    return plsc.load_gather(ref1d, [jnp.full((16,), idx, jnp.int32)])


def _merge16(ak, av, bk, bv):
    rbk = lax.rev(bk, (0,))
    rbv = lax.rev(bv, (0,))
    m = (ak < rbk) | ((ak == rbk) & (av <= rbv))
    mk = jnp.where(m, ak, rbk)
    mv = jnp.where(m, av, rbv)
    sk, sv = plsc.sort_key_val(mk, mv)
    return sk, sv


def _tie_fix(k, v, partner, iota):
    kp = k.at[partner].get(mode='promise_in_bounds')
    vp = v.at[partner].get(mode='promise_in_bounds')
    tie = k == kp
    nv = jnp.where(tie & (iota < partner), jnp.minimum(v, vp),
                   jnp.where(tie & (iota > partner), jnp.maximum(v, vp), v))
    return nv


def _sc_select_gather(bm4, d2r, xyzT, coordflat, fvecf, BM, N):
    mesh = plsc.VectorSubcoreMesh(core_axis_name="c", subcore_axis_name="s")

    @functools.partial(
        pl.kernel,
        out_type=[
            jax.ShapeDtypeStruct((BM * K, 256), jnp.float32),
            jax.ShapeDtypeStruct((BM * K * 16,), jnp.float32),
        ],
        mesh=mesh,
        compiler_params=pltpu.CompilerParams(needs_layout_passes=False),
        scratch_types=[
            pltpu.VMEM((8 * (QPW // 2) * 16,), jnp.float32),
            pltpu.VMEM((QPW * 3,), jnp.float32),
            pltpu.VMEM((QPW * K,), jnp.int32),
            pltpu.SMEM((QPW,), jnp.float32),
            pltpu.VMEM((2, ROWS, BLK), jnp.float32),
            pltpu.VMEM((ROWS,), jnp.int32),
            pltpu.VMEM((ROWS, 256), jnp.float32),
            pltpu.VMEM((N,), jnp.float32),
            pltpu.VMEM((N,), jnp.float32),
            pltpu.VMEM((N,), jnp.float32),
            pltpu.VMEM((ROWS * 16,), jnp.float32),
            pltpu.SemaphoreType.DMA((2,)),
            pltpu.SemaphoreType.DMA,
        ],
    )
    def body(bm_hbm, d2r_hbm, xyzT_hbm, cp_hbm, fv_hbm, gath_hbm, xout_hbm,
             bmv, cpflat, rowflat, thrv, candv, fidxv, fbufv, px, py, pz, xov,
             cand_sem, fg_sem):
        wid = lax.axis_index("s") * NC + lax.axis_index("c")
        qbase = wid * QPW
        b = qbase // 2048
        mq = qbase - b * 2048
        nbase = b * N
        iota = lax.iota(jnp.int32, 16)

        pltpu.sync_copy(cp_hbm.at[pl.ds(qbase * 3, QPW * 3)], cpflat)
        pltpu.sync_copy(xyzT_hbm.at[pl.ds((b * 3 + 0) * N, N)], px)
        pltpu.sync_copy(xyzT_hbm.at[pl.ds((b * 3 + 1) * N, N)], py)
        pltpu.sync_copy(xyzT_hbm.at[pl.ds((b * 3 + 2) * N, N)], pz)

        for h in range(2):
            hq = h * (QPW // 2)
            for r in range(8):
                pltpu.sync_copy(
                    bm_hbm.at[pl.ds(((b * 8 + r) * 2048 + mq + hq) * 16,
                                    (QPW // 2) * 16)],
                    bmv.at[pl.ds(r * (QPW // 2) * 16, (QPW // 2) * 16)])

            def p1(q, _, hq=hq):
                lq = hq + q
                ks, vs = [], []
                for r in range(8):
                    k = bmv[pl.ds((r * (QPW // 2) + q) * 16, 16)]
                    k, v = plsc.sort_key_val(k, iota + r * 16)
                    ks.append(k)
                    vs.append(v)
                while len(ks) > 1:
                    nk, nv = [], []
                    for i in range(0, len(ks), 2):
                        mk, mv = _merge16(ks[i], vs[i], ks[i + 1], vs[i + 1])
                        nk.append(mk)
                        nv.append(mv)
                    ks, vs = nk, nv
                thrv[lq] = jnp.max(ks[0])
                sid, _ = plsc.sort_key_val(vs[0], vs[0])
                rowflat[pl.ds(lq * 16, 16)] = sid + (qbase + lq) * NB
                return 0

            lax.fori_loop(0, QPW // 2, p1, 0)

        def cand_fetch(chunk, slot):
            return pltpu.async_copy(
                d2r_hbm.at[rowflat.at[pl.ds(chunk * ROWS, ROWS)]],
                candv.at[slot], cand_sem.at[slot])

        cand_fetch(0, 0)
        cand_fetch(1, 1)

        def chunk_body(p, _):
            for s in range(2):
                c = p * 2 + s
                pltpu.make_async_copy(
                    d2r_hbm.at[pl.ds(0, ROWS)], candv.at[s],
                    cand_sem.at[s]).wait()

                def q_body(q2, _):
                    lq = c * CQ + q2
                    theta = thrv[lq]
                    qg128 = (qbase + lq) * NB

                    def bi_body(bi, cur):
                        ck, cv = cur
                        blkb = plsc.load_gather(
                            rowflat,
                            [jnp.full((16,), lq * 16 + bi, jnp.int32)])
                        row = q2 * 16 + bi
                        for part in range(8):
                            v = candv[s, row, pl.ds(part * 16, 16)]
                            msk = v <= theta
                            vidx = (blkb - qg128) * BLK + part * 16 + iota

                            def do(ck, cv, v, vidx, msk):
                                sk, sv = plsc.sort_key_val(
                                    jnp.where(msk, v, INF), vidx)
                                return _merge16(ck, cv, sk, sv)

                            ck, cv = lax.cond(
                                jnp.any(msk), do,
                                lambda ck, cv, v, vidx, msk: (ck, cv),
                                ck, cv, v, vidx, msk)
                        return ck, cv

                    fin_k, fin_v = lax.fori_loop(
                        0, 16, bi_body,
                        (jnp.full((16,), INF, jnp.float32),
                         jnp.zeros((16,), jnp.int32)))
                    p_even = lax.bitwise_xor(iota, 1)
                    p_odd = jnp.clip(lax.bitwise_xor(iota - 1, 1) + 1, 0, 15)
                    for _ in range(2):
                        fin_v = _tie_fix(fin_k, fin_v, p_even, iota)
                        fin_v = _tie_fix(fin_k, fin_v, p_odd, iota)
                    fidxv[pl.ds(q2 * 16, 16)] = fin_v + nbase
                    pltpu.async_copy(
                        fv_hbm.at[fidxv.at[pl.ds(q2 * 16, 16)]],
                        fbufv.at[pl.ds(q2 * 16, 16)], fg_sem)

                    nx = plsc.load_gather(px, [fin_v])
                    ny = plsc.load_gather(py, [fin_v])
                    nz = plsc.load_gather(pz, [fin_v])
                    rowv = (q2 * 16 + iota) * 16
                    cols = [nx - _bcast(cpflat, lq * 3 + 0),
                            ny - _bcast(cpflat, lq * 3 + 1),
                            nz - _bcast(cpflat, lq * 3 + 2),
                            nx, ny, nz]
                    for j, colv in enumerate(cols):
                        plsc.store_scatter(xov, [rowv + j], colv)
                    return 0

                lax.fori_loop(0, CQ, q_body, 0)

                obase = (qbase + c * CQ) * K
                for _ in range(CQ):
                    pltpu.make_async_copy(
                        fv_hbm.at[pl.ds(0, 16)],
                        fbufv.at[pl.ds(0, 16)], fg_sem).wait()
                pltpu.sync_copy(fbufv, gath_hbm.at[pl.ds(obase, ROWS)])
                pltpu.sync_copy(xov, xout_hbm.at[pl.ds(obase * 16, ROWS * 16)])

                cand_fetch(jnp.minimum(c + 2, NCHUNK - 1), s)
            return 0

        lax.fori_loop(0, NCHUNK // 2, chunk_body, 0)
        for s in range(2):
            pltpu.make_async_copy(
                d2r_hbm.at[pl.ds(0, ROWS)], candv.at[s], cand_sem.at[s]).wait()

    return body(bm4, d2r, xyzT, coordflat, fvecf)


def _corr_body(g_ref, t_ref, p_ref, x_ref, o_ref):
    g = g_ref[...]
    t = t_ref[...]
    tq = t.shape[0]
    trep = jnp.broadcast_to(t[:, None, :], (tq, K, 256)).reshape(tq * K, 256)
    z = g * trep
    corr = jnp.dot(z, p_ref[...], preferred_element_type=jnp.float32) \
        / np.float32((256.0 / GROUPS) ** 0.5)
    o_ref[...] = jnp.concatenate([corr, x_ref[...][:, :6]], axis=1)


def _corr(gath, targets_flat, pmask, xout):
    R = gath.shape[0]
    TQ = 128
    return pl.pallas_call(
        _corr_body,
        grid=(R // (TQ * K),),
        in_specs=[
            pl.BlockSpec((TQ * K, 256), lambda i: (i, 0)),
            pl.BlockSpec((TQ, 256), lambda i: (i, 0)),
            pl.BlockSpec((256, GROUPS), lambda i: (0, 0)),
            pl.BlockSpec((TQ * K, 16), lambda i: (i, 0)),
        ],
        out_specs=pl.BlockSpec((TQ * K, GROUPS + 6), lambda i: (i, 0)),
        out_shape=jax.ShapeDtypeStruct((R, GROUPS + 6), jnp.float32),
    )(gath, targets_flat, pmask, xout)


def kernel(xyz, fvec, targets, coords_world_xyz):
    B, N, C = fvec.shape
    M = targets.shape[1]
    BM = B * M

    xyzT = jnp.transpose(xyz, (0, 2, 1))
    d2, bmins4 = _d2_and_blockmins(coords_world_xyz, xyz)

    d2r = d2.reshape(BM * NB, BLK)
    coordflat = coords_world_xyz.reshape(BM * 3)
    fvecf = fvec.reshape(B * N, C)

    xyzTflat = xyzT.reshape(B * 3 * N)
    bmflat = bmins4.reshape(B * 8 * M * 16)
    gath, xoutf = _sc_select_gather(bmflat, d2r, xyzTflat, coordflat, fvecf,
                                    BM, N)

    pmask = jnp.asarray(
        (np.arange(256)[:, None] // (C // GROUPS)
         == np.arange(GROUPS)[None, :]).astype(np.float32))
    out = _corr(gath, targets.reshape(BM, C), pmask,
                xoutf.reshape(BM * K, 16))
    return out.reshape(B, M, K, GROUPS + 6)


# speedup vs baseline: 1.5068x; 1.0128x over previous
import functools

import jax
import jax.numpy as jnp
import numpy as np
from jax import lax
from jax.experimental import pallas as pl
from jax.experimental.pallas import tpu as pltpu
from jax.experimental.pallas import tpu_sc as plsc

K = 16
GROUPS = 8
BLK = 128
NB = 128
NC, NS = 2, 16
NW = NC * NS
QPW = 128
CQ = 4
ROWS = CQ * K
NCHUNK = QPW // CQ
INF = np.float32(np.inf)


def _d2_body(q_ref, x_ref, d2_ref, bm_ref):
    q = q_ref[0]
    x = x_ref[0]
    qn = jnp.sum(q * q, axis=1)
    pn = jnp.sum(x * x, axis=1)
    cross = lax.dot_general(q, x, (((1,), (1,)), ((), ())),
                            preferred_element_type=jnp.float32)
    d2 = qn[:, None] + pn[None, :] - 2.0 * cross
    tm, tn = d2.shape
    d2b = d2.reshape(tm, tn // BLK, BLK)
    d2_ref[0] = d2b
    bm_ref[0, 0] = jnp.min(d2b, axis=-1).reshape(tm // 8, 8 * (tn // BLK))


def _d2_and_blockmins(coords, xyz):
    B, M, _ = coords.shape
    N = xyz.shape[1]
    TM, TN = 256, 2048
    return pl.pallas_call(
        _d2_body,
        grid=(B, M // TM, N // TN),
        in_specs=[
            pl.BlockSpec((1, TM, 3), lambda b, i, j: (b, i, 0)),
            pl.BlockSpec((1, TN, 3), lambda b, i, j: (b, j, 0)),
        ],
        out_specs=[
            pl.BlockSpec((1, TM, TN // BLK, BLK), lambda b, i, j: (b, i, j, 0)),
            pl.BlockSpec((1, 1, TM // 8, 8 * (TN // BLK)),
                         lambda b, i, j: (b, j, i, 0)),
        ],
        out_shape=[
            jax.ShapeDtypeStruct((B, M, N // BLK, BLK), jnp.float32),
            jax.ShapeDtypeStruct((B, N // TN, M // 8, 8 * (TN // BLK)),
                                 jnp.float32),
        ],
    )(coords, xyz)


def _bcast(ref1d, idx):
    return plsc.load_gather(ref1d, [jnp.full((16,), idx, jnp.int32)])


def _merge16(ak, av, bk, bv):
    rbk = lax.rev(bk, (0,))
    rbv = lax.rev(bv, (0,))
    m = (ak < rbk) | ((ak == rbk) & (av <= rbv))
    mk = jnp.where(m, ak, rbk)
    mv = jnp.where(m, av, rbv)
    sk, sv = plsc.sort_key_val(mk, mv)
    return sk, sv


def _tie_fix(k, v, partner, iota):
    kp = k.at[partner].get(mode='promise_in_bounds')
    vp = v.at[partner].get(mode='promise_in_bounds')
    tie = k == kp
    nv = jnp.where(tie & (iota < partner), jnp.minimum(v, vp),
                   jnp.where(tie & (iota > partner), jnp.maximum(v, vp), v))
    return nv


def _sc_select_gather(bm4, d2r, xyzT, coordflat, fvecf, BM, N):
    mesh = plsc.VectorSubcoreMesh(core_axis_name="c", subcore_axis_name="s")

    @functools.partial(
        pl.kernel,
        out_type=[
            jax.ShapeDtypeStruct((BM * K, 256), jnp.float32),
            jax.ShapeDtypeStruct((BM * K, 128), jnp.float32),
        ],
        mesh=mesh,
        compiler_params=pltpu.CompilerParams(needs_layout_passes=False),
        scratch_types=[
            pltpu.VMEM((8 * (QPW // 2) * 16,), jnp.float32),
            pltpu.VMEM((QPW * 3,), jnp.float32),
            pltpu.VMEM((QPW * K,), jnp.int32),
            pltpu.SMEM((QPW,), jnp.float32),
            pltpu.VMEM((2, ROWS, BLK), jnp.float32),
            pltpu.VMEM((ROWS,), jnp.int32),
            pltpu.VMEM((ROWS, 256), jnp.float32),
            pltpu.VMEM((N,), jnp.float32),
            pltpu.VMEM((N,), jnp.float32),
            pltpu.VMEM((N,), jnp.float32),
            pltpu.VMEM((ROWS, 128), jnp.float32),
            pltpu.SemaphoreType.DMA((2,)),
            pltpu.SemaphoreType.DMA,
        ],
    )
    def body(bm_hbm, d2r_hbm, xyzT_hbm, cp_hbm, fv_hbm, gath_hbm, xout_hbm,
             bmv, cpflat, rowflat, thrv, candv, fidxv, fbufv, px, py, pz, xov,
             cand_sem, fg_sem):
        wid = lax.axis_index("s") * NC + lax.axis_index("c")
        qbase = wid * QPW
        b = qbase // 2048
        mq = qbase - b * 2048
        nbase = b * N
        iota = lax.iota(jnp.int32, 16)

        pltpu.sync_copy(cp_hbm.at[pl.ds(qbase * 3, QPW * 3)], cpflat)
        pltpu.sync_copy(xyzT_hbm.at[pl.ds((b * 3 + 0) * N, N)], px)
        pltpu.sync_copy(xyzT_hbm.at[pl.ds((b * 3 + 1) * N, N)], py)
        pltpu.sync_copy(xyzT_hbm.at[pl.ds((b * 3 + 2) * N, N)], pz)

        for h in range(2):
            hq = h * (QPW // 2)
            for r in range(8):
                pltpu.sync_copy(
                    bm_hbm.at[pl.ds(((b * 8 + r) * 2048 + mq + hq) * 16,
                                    (QPW // 2) * 16)],
                    bmv.at[pl.ds(r * (QPW // 2) * 16, (QPW // 2) * 16)])

            def p1(q, _, hq=hq):
                lq = hq + q
                ks, vs = [], []
                for r in range(8):
                    k = bmv[pl.ds((r * (QPW // 2) + q) * 16, 16)]
                    k, v = plsc.sort_key_val(k, iota + r * 16)
                    ks.append(k)
                    vs.append(v)
                while len(ks) > 1:
                    nk, nv = [], []
                    for i in range(0, len(ks), 2):
                        mk, mv = _merge16(ks[i], vs[i], ks[i + 1], vs[i + 1])
                        nk.append(mk)
                        nv.append(mv)
                    ks, vs = nk, nv
                thrv[lq] = jnp.max(ks[0])
                sid, _ = plsc.sort_key_val(vs[0], vs[0])
                rowflat[pl.ds(lq * 16, 16)] = sid + (qbase + lq) * NB
                return 0

            lax.fori_loop(0, QPW // 2, p1, 0)

        def cand_fetch(chunk, slot):
            return pltpu.async_copy(
                d2r_hbm.at[rowflat.at[pl.ds(chunk * ROWS, ROWS)]],
                candv.at[slot], cand_sem.at[slot])

        cand_fetch(0, 0)
        cand_fetch(1, 1)

        def chunk_body(p, _):
            for s in range(2):
                c = p * 2 + s
                pltpu.make_async_copy(
                    d2r_hbm.at[pl.ds(0, ROWS)], candv.at[s],
                    cand_sem.at[s]).wait()

                def q_body(q2, _):
                    lq = c * CQ + q2
                    theta = thrv[lq]
                    qg128 = (qbase + lq) * NB

                    def bi_body(bi, cur):
                        ck, cv = cur
                        blkb = plsc.load_gather(
                            rowflat,
                            [jnp.full((16,), lq * 16 + bi, jnp.int32)])
                        row = q2 * 16 + bi
                        for part in range(8):
                            v = candv[s, row, pl.ds(part * 16, 16)]
                            msk = v <= theta
                            vidx = (blkb - qg128) * BLK + part * 16 + iota

                            def do(ck, cv, v, vidx, msk):
                                sk, sv = plsc.sort_key_val(
                                    jnp.where(msk, v, INF), vidx)
                                return _merge16(ck, cv, sk, sv)

                            ck, cv = lax.cond(
                                jnp.any(msk), do,
                                lambda ck, cv, v, vidx, msk: (ck, cv),
                                ck, cv, v, vidx, msk)
                        return ck, cv

                    fin_k, fin_v = lax.fori_loop(
                        0, 16, bi_body,
                        (jnp.full((16,), INF, jnp.float32),
                         jnp.zeros((16,), jnp.int32)))
                    p_even = lax.bitwise_xor(iota, 1)
                    p_odd = jnp.clip(lax.bitwise_xor(iota - 1, 1) + 1, 0, 15)
                    for _ in range(2):
                        fin_v = _tie_fix(fin_k, fin_v, p_even, iota)
                        fin_v = _tie_fix(fin_k, fin_v, p_odd, iota)
                    fidxv[pl.ds(q2 * 16, 16)] = fin_v + nbase
                    pltpu.async_copy(
                        fv_hbm.at[fidxv.at[pl.ds(q2 * 16, 16)]],
                        fbufv.at[pl.ds(q2 * 16, 16)], fg_sem)

                    nx = plsc.load_gather(px, [fin_v])
                    ny = plsc.load_gather(py, [fin_v])
                    nz = plsc.load_gather(pz, [fin_v])
                    rowv = q2 * 16 + iota
                    cols = [nx - _bcast(cpflat, lq * 3 + 0),
                            ny - _bcast(cpflat, lq * 3 + 1),
                            nz - _bcast(cpflat, lq * 3 + 2),
                            nx, ny, nz]
                    for j, colv in enumerate(cols):
                        plsc.store_scatter(
                            xov, [rowv, jnp.full((16,), j, jnp.int32)], colv)
                    return 0

                lax.fori_loop(0, CQ, q_body, 0)

                obase = (qbase + c * CQ) * K
                for _ in range(CQ):
                    pltpu.make_async_copy(
                        fv_hbm.at[pl.ds(0, 16)],
                        fbufv.at[pl.ds(0, 16)], fg_sem).wait()
                pltpu.sync_copy(fbufv, gath_hbm.at[pl.ds(obase, ROWS)])
                pltpu.sync_copy(xov, xout_hbm.at[pl.ds(obase, ROWS)])

                cand_fetch(jnp.minimum(c + 2, NCHUNK - 1), s)
            return 0

        lax.fori_loop(0, NCHUNK // 2, chunk_body, 0)
        for s in range(2):
            pltpu.make_async_copy(
                d2r_hbm.at[pl.ds(0, ROWS)], candv.at[s], cand_sem.at[s]).wait()

    return body(bm4, d2r, xyzT, coordflat, fvecf)


def _corr_body(g_ref, t_ref, p_ref, x_ref, o_ref):
    g = g_ref[...]
    t = t_ref[...]
    tq = t.shape[0]
    trep = jnp.broadcast_to(t[:, None, :], (tq, K, 256)).reshape(tq * K, 256)
    z = g * trep
    corr = jnp.dot(z, p_ref[...], preferred_element_type=jnp.float32) \
        / np.float32((256.0 / GROUPS) ** 0.5)
    o_ref[...] = jnp.concatenate([corr, x_ref[...][:, :6]], axis=1)


def _corr(gath, targets_flat, pmask, xout):
    R = gath.shape[0]
    TQ = 128
    return pl.pallas_call(
        _corr_body,
        grid=(R // (TQ * K),),
        in_specs=[
            pl.BlockSpec((TQ * K, 256), lambda i: (i, 0)),
            pl.BlockSpec((TQ, 256), lambda i: (i, 0)),
            pl.BlockSpec((256, GROUPS), lambda i: (0, 0)),
            pl.BlockSpec((TQ * K, 128), lambda i: (i, 0)),
        ],
        out_specs=pl.BlockSpec((TQ * K, GROUPS + 6), lambda i: (i, 0)),
        out_shape=jax.ShapeDtypeStruct((R, GROUPS + 6), jnp.float32),
    )(gath, targets_flat, pmask, xout)


def kernel(xyz, fvec, targets, coords_world_xyz):
    B, N, C = fvec.shape
    M = targets.shape[1]
    BM = B * M

    xyzT = jnp.transpose(xyz, (0, 2, 1))
    d2, bmins4 = _d2_and_blockmins(coords_world_xyz, xyz)

    d2r = d2.reshape(BM * NB, BLK)
    coordflat = coords_world_xyz.reshape(BM * 3)
    fvecf = fvec.reshape(B * N, C)

    xyzTflat = xyzT.reshape(B * 3 * N)
    bmflat = bmins4.reshape(B * 8 * M * 16)
    gath, xoutf = _sc_select_gather(bmflat, d2r, xyzTflat, coordflat, fvecf,
                                    BM, N)

    pmask = jnp.asarray(
        (np.arange(256)[:, None] // (C // GROUPS)
         == np.arange(GROUPS)[None, :]).astype(np.float32))
    out = _corr(gath, targets.reshape(BM, C), pmask, xoutf)
    return out.reshape(B, M, K, GROUPS + 6)
